# baseline (speedup 1.0000x reference)
import functools

import jax
import jax.numpy as jnp
from jax import lax
from jax.experimental import pallas as pl
from jax.experimental.pallas import tpu as pltpu
from jax.experimental.pallas import tpu_sc as plsc

_NC = 2
_NS = 16
_NW = _NC * _NS


def _sc_gather(V, lin_table, idx_wm, M, B, nbuf=4):
    K = V.shape[1]
    chunk = B // _NW
    lin_flat = lin_table.reshape(-1)
    assert chunk % 8 == 0

    mesh = plsc.VectorSubcoreMesh(
        core_axis_name="c", subcore_axis_name="s",
        num_cores=_NC, num_subcores=_NS,
    )

    @functools.partial(
        pl.kernel,
        mesh=mesh,
        compiler_params=pltpu.CompilerParams(use_tc_tiling_on_sc=False),
        out_type=(
            jax.ShapeDtypeStruct((M, B, K), jnp.float32),
            jax.ShapeDtypeStruct((M, _NW, chunk), jnp.float32),
        ),
        scratch_types=[
            pltpu.VMEM((M, chunk), jnp.int32),
            pltpu.VMEM((M, chunk), jnp.float32),
            pltpu.VMEM((nbuf, chunk, K), jnp.float32),
            pltpu.SemaphoreType.DMA,
            pltpu.SemaphoreType.DMA,
            pltpu.SemaphoreType.DMA,
        ],
    )
    def gather_kernel(v_hbm, lin_hbm, idx_hbm, emb_out, lin_out,
                      idx_v, lin_v, rows_v, sem_g, sem_o, sem_l):
        wid = lax.axis_index("s") * _NC + lax.axis_index("c")
        pltpu.sync_copy(idx_hbm.at[wid], idx_v)
        for m in range(M):
            pltpu.async_copy(lin_hbm.at[idx_v.at[m]], lin_v.at[m], sem_l)

        def fire(m):
            pltpu.async_copy(v_hbm.at[idx_v.at[m]], rows_v.at[m % nbuf],
                             sem_g)

        def wait_gather(m):
            pltpu.make_async_copy(v_hbm.at[idx_v.at[m]],
                                  rows_v.at[m % nbuf], sem_g).wait()

        def copy_out(m):
            pltpu.async_copy(rows_v.at[m % nbuf],
                             emb_out.at[m, pl.ds(wid * chunk, chunk)],
                             sem_o)

        def wait_out(m):
            pltpu.make_async_copy(rows_v.at[m % nbuf],
                                  emb_out.at[m, pl.ds(wid * chunk, chunk)],
                                  sem_o).wait()

        prefire = nbuf - 1
        for m in range(prefire):
            fire(m)
        for m in range(M):
            wait_gather(m)
            copy_out(m)
            f = m + prefire
            if f < M:
                if f >= nbuf:
                    wait_out(f - nbuf)
                fire(f)
        for m in range(max(M - nbuf, 0), M):
            wait_out(m)
        for m in range(M):
            pltpu.make_async_copy(lin_hbm.at[idx_v.at[m]], lin_v.at[m],
                                  sem_l).wait()
        pltpu.sync_copy(lin_v, lin_out.at[pl.ds(0, M), wid])

    return gather_kernel(V, lin_flat, idx_wm)


def _mlp_block(e3_ref, dense_ref, linv_ref, w0_ref, w0d_ref, b0_ref,
               w1_ref, b1_ref, w2_ref, b2_ref, wh_ref, wli_ref, out_ref,
               acc_ref, s_ref, ss_ref,
               *, m_fields, k_dim, mpj, nj):
    j = pl.program_id(1)

    planes = [e3_ref[m] for m in range(mpj)]
    e = jnp.concatenate(planes, axis=1)
    part = e @ w0_ref[...]
    ps = planes[0]
    for m in range(1, mpj):
        ps = ps + planes[m]
    pss = jnp.sum(e * e, axis=1, keepdims=True)

    @pl.when(j == 0)
    def _init():
        acc_ref[...] = part
        s_ref[...] = ps
        ss_ref[...] = pss

    @pl.when(j > 0)
    def _accum():
        acc_ref[...] += part
        s_ref[...] += ps
        ss_ref[...] += pss

    @pl.when(j == nj - 1)
    def _tail():
        s = s_ref[...]
        inter = 0.5 * (jnp.sum(s * s, axis=1, keepdims=True) - ss_ref[...])
        lin = jnp.sum(linv_ref[...], axis=1, keepdims=True)
        h = (acc_ref[...] + dense_ref[...] @ w0d_ref[...] + b0_ref[...])
        h = jnp.maximum(h, 0.0)
        h = jnp.maximum(h @ w1_ref[...] + b1_ref[...], 0.0)
        h = jnp.maximum(h @ w2_ref[...] + b2_ref[...], 0.0)
        wli = wli_ref[...]
        z = (h @ wh_ref[...] + lin * wli[0, 0] + inter * wli[0, 1]
             + wli[0, 2])
        out_ref[...] = jax.nn.sigmoid(z)


def _tc_mlp(e3, dense, linv, W0, b0, W1, b1, W2, b2, Wfc, bfc,
            block_b, mpj=2, interpret=False):
    M, B, K = e3.shape
    D = dense.shape[1]
    H0, H1, H2 = W0.shape[1], W1.shape[1], W2.shape[1]
    nj = M // mpj
    wh = Wfc[2:]
    wli = jnp.concatenate([Wfc[0:1, 0], Wfc[1:2, 0], bfc]).reshape(1, 3)
    w0d = W0[M * K:]
    grid = (B // block_b, nj)

    out = pl.pallas_call(
        functools.partial(_mlp_block, m_fields=M, k_dim=K, mpj=mpj, nj=nj),
        grid=grid,
        in_specs=[
            pl.BlockSpec((mpj, block_b, K), lambda i, j: (j, i, 0)),
            pl.BlockSpec((block_b, D), lambda i, j: (i, 0)),
            pl.BlockSpec((block_b, M), lambda i, j: (i, 0)),
            pl.BlockSpec((mpj * K, H0), lambda i, j: (j, 0)),
            pl.BlockSpec((D, H0), lambda i, j: (0, 0)),
            pl.BlockSpec((1, H0), lambda i, j: (0, 0)),
            pl.BlockSpec((H0, H1), lambda i, j: (0, 0)),
            pl.BlockSpec((1, H1), lambda i, j: (0, 0)),
            pl.BlockSpec((H1, H2), lambda i, j: (0, 0)),
            pl.BlockSpec((1, H2), lambda i, j: (0, 0)),
            pl.BlockSpec((H2, 1), lambda i, j: (0, 0)),
            pl.BlockSpec((1, 3), lambda i, j: (0, 0)),
        ],
        out_specs=pl.BlockSpec((block_b, 1), lambda i, j: (i, 0)),
        out_shape=jax.ShapeDtypeStruct((B, 1), jnp.float32),
        scratch_shapes=[
            pltpu.VMEM((block_b, H0), jnp.float32),
            pltpu.VMEM((block_b, K), jnp.float32),
            pltpu.VMEM((block_b, 1), jnp.float32),
        ],
        interpret=interpret,
    )(e3, dense, linv, W0, w0d, b0.reshape(1, H0), W1,
      b1.reshape(1, H1), W2, b2.reshape(1, H2), wh, wli)
    return out[:, 0]


def kernel(cat_features, dense_features, lin_table, V, W0, b0, W1, b1,
           W2, b2, Wfc, bfc):
    B, M = cat_features.shape
    K = V.shape[1]
    sizes = (2048, 2048)
    cat32 = cat_features.astype(jnp.int32)
    outs = []
    off = 0
    for Bs in sizes:
        chunk = Bs // _NW
        cat_p = lax.slice_in_dim(cat32, off, off + Bs)
        idx_wm = cat_p.reshape(_NW, chunk, M).transpose(0, 2, 1)
        e3, lin_mw = _sc_gather(V, lin_table, idx_wm, M, Bs)
        linv = lin_mw.reshape(M, Bs).T
        dense_p = lax.slice_in_dim(dense_features, off, off + Bs)
        outs.append(_tc_mlp(e3, dense_p, linv, W0, b0, W1, b1, W2,
                            b2, Wfc, bfc, block_b=512))
        off += Bs
    return jnp.concatenate(outs, axis=0)


# speedup vs baseline: 1.6549x; 1.6549x over previous
import functools

import jax
import jax.numpy as jnp
from jax import lax
from jax.experimental import pallas as pl
from jax.experimental.pallas import tpu as pltpu
from jax.experimental.pallas import tpu_sc as plsc

_NC = 2
_NS = 16
_NW = _NC * _NS


def _sc_gather(V, lin_table, idx_wm, M, B, nbuf=4):
    K = V.shape[1]
    chunk = B // _NW
    lin_flat = lin_table.reshape(-1)
    assert chunk % 8 == 0

    mesh = plsc.VectorSubcoreMesh(
        core_axis_name="c", subcore_axis_name="s",
        num_cores=_NC, num_subcores=_NS,
    )

    @functools.partial(
        pl.kernel,
        mesh=mesh,
        compiler_params=pltpu.CompilerParams(use_tc_tiling_on_sc=False),
        out_type=(
            jax.ShapeDtypeStruct((M, B, K), jnp.float32),
            jax.ShapeDtypeStruct((M, _NW, chunk), jnp.float32),
        ),
        scratch_types=[
            pltpu.VMEM((M, chunk), jnp.int32),
            pltpu.VMEM((M, chunk), jnp.float32),
            pltpu.VMEM((nbuf, chunk, K), jnp.float32),
            pltpu.SemaphoreType.DMA,
            pltpu.SemaphoreType.DMA,
            pltpu.SemaphoreType.DMA,
        ],
    )
    def gather_kernel(v_hbm, lin_hbm, idx_hbm, emb_out, lin_out,
                      idx_v, lin_v, rows_v, sem_g, sem_o, sem_l):
        wid = lax.axis_index("s") * _NC + lax.axis_index("c")
        pltpu.sync_copy(idx_hbm.at[wid], idx_v)
        for m in range(M):
            pltpu.async_copy(lin_hbm.at[idx_v.at[m]], lin_v.at[m], sem_l)

        def fire(m):
            pltpu.async_copy(v_hbm.at[idx_v.at[m]], rows_v.at[m % nbuf],
                             sem_g)

        def wait_gather(m):
            pltpu.make_async_copy(v_hbm.at[idx_v.at[m]],
                                  rows_v.at[m % nbuf], sem_g).wait()

        def copy_out(m):
            pltpu.async_copy(rows_v.at[m % nbuf],
                             emb_out.at[m, pl.ds(wid * chunk, chunk)],
                             sem_o)

        def wait_out(m):
            pltpu.make_async_copy(rows_v.at[m % nbuf],
                                  emb_out.at[m, pl.ds(wid * chunk, chunk)],
                                  sem_o).wait()

        prefire = nbuf - 1
        for m in range(prefire):
            fire(m)
        for m in range(M):
            wait_gather(m)
            copy_out(m)
            f = m + prefire
            if f < M:
                if f >= nbuf:
                    wait_out(f - nbuf)
                fire(f)
        for m in range(max(M - nbuf, 0), M):
            wait_out(m)
        for m in range(M):
            pltpu.make_async_copy(lin_hbm.at[idx_v.at[m]], lin_v.at[m],
                                  sem_l).wait()
        pltpu.sync_copy(lin_v, lin_out.at[pl.ds(0, M), wid])

    return gather_kernel(V, lin_flat, idx_wm)


def _mlp_block(e3_ref, dense_ref, linv_ref, w0_ref, b0_ref,
               w1_ref, b1_ref, w2_ref, b2_ref, wh_ref, wli_ref, out_ref,
               *, m_fields, k_dim):
    planes = [e3_ref[m] for m in range(m_fields)]
    s = planes[0]
    ss = jnp.sum(planes[0] * planes[0], axis=1, keepdims=True)
    for m in range(1, m_fields):
        p = planes[m]
        s = s + p
        ss = ss + jnp.sum(p * p, axis=1, keepdims=True)
    inter = 0.5 * (jnp.sum(s * s, axis=1, keepdims=True) - ss)
    lin = jnp.sum(linv_ref[...], axis=1, keepdims=True)

    e = jnp.concatenate(planes, axis=1)
    mk = m_fields * k_dim
    h = (e @ w0_ref[0:mk, :] + dense_ref[...] @ w0_ref[mk:, :]
         + b0_ref[...])
    h = jnp.maximum(h, 0.0)
    h = jnp.maximum(h @ w1_ref[...] + b1_ref[...], 0.0)
    h = jnp.maximum(h @ w2_ref[...] + b2_ref[...], 0.0)
    wli = wli_ref[...]
    z = (h @ wh_ref[...] + lin * wli[0, 0] + inter * wli[0, 1]
         + wli[0, 2])
    out_ref[...] = jax.nn.sigmoid(z)


def _tc_mlp(e3, dense, linv, W0, b0, W1, b1, W2, b2, Wfc, bfc,
            block_b, interpret=False):
    M, B, K = e3.shape
    D = dense.shape[1]
    H0, H1, H2 = W0.shape[1], W1.shape[1], W2.shape[1]
    wh = Wfc[2:]
    wli = jnp.concatenate([Wfc[0:1, 0], Wfc[1:2, 0], bfc]).reshape(1, 3)
    grid = (B // block_b,)

    out = pl.pallas_call(
        functools.partial(_mlp_block, m_fields=M, k_dim=K),
        grid=grid,
        in_specs=[
            pl.BlockSpec((M, block_b, K), lambda i: (0, i, 0)),
            pl.BlockSpec((block_b, D), lambda i: (i, 0)),
            pl.BlockSpec((block_b, M), lambda i: (i, 0)),
            pl.BlockSpec((M * K + D, H0), lambda i: (0, 0)),
            pl.BlockSpec((1, H0), lambda i: (0, 0)),
            pl.BlockSpec((H0, H1), lambda i: (0, 0)),
            pl.BlockSpec((1, H1), lambda i: (0, 0)),
            pl.BlockSpec((H1, H2), lambda i: (0, 0)),
            pl.BlockSpec((1, H2), lambda i: (0, 0)),
            pl.BlockSpec((H2, 1), lambda i: (0, 0)),
            pl.BlockSpec((1, 3), lambda i: (0, 0)),
        ],
        out_specs=pl.BlockSpec((block_b, 1), lambda i: (i, 0)),
        out_shape=jax.ShapeDtypeStruct((B, 1), jnp.float32),
        interpret=interpret,
    )(e3, dense, linv, W0, b0.reshape(1, H0), W1,
      b1.reshape(1, H1), W2, b2.reshape(1, H2), wh, wli)
    return out[:, 0]


def kernel(cat_features, dense_features, lin_table, V, W0, b0, W1, b1,
           W2, b2, Wfc, bfc):
    B, M = cat_features.shape
    K = V.shape[1]
    sizes = (2048, 2048)
    cat32 = cat_features.astype(jnp.int32)
    outs = []
    off = 0
    for Bs in sizes:
        chunk = Bs // _NW
        cat_p = lax.slice_in_dim(cat32, off, off + Bs)
        idx_wm = cat_p.reshape(_NW, chunk, M).transpose(0, 2, 1)
        e3, lin_mw = _sc_gather(V, lin_table, idx_wm, M, Bs)
        linv = lin_mw.reshape(M, Bs).T
        dense_p = lax.slice_in_dim(dense_features, off, off + Bs)
        outs.append(_tc_mlp(e3, dense_p, linv, W0, b0, W1, b1, W2,
                            b2, Wfc, bfc, block_b=512))
        off += Bs
    return jnp.concatenate(outs, axis=0)
